# baseline (speedup 1.0000x reference)
import functools

import jax
import jax.numpy as jnp
import numpy as np
from jax import lax
from jax.experimental import pallas as pl
from jax.experimental.pallas import tpu as pltpu
from jax.experimental.pallas import tpu_sc as plsc

VOCAB = 100000
FACTOR = 64
HIDDEN = 768
BATCH = 4
SEQ = 2048
NTOK = BATCH * SEQ

_NC, _NS = 2, 16
_NW = _NC * _NS
_B_PER_W = NTOK // _NW


def _sinusoidal_pe(seq_len, dim):
    pos = np.arange(seq_len)[:, None].astype(np.float32)
    i = np.arange(dim)[None, :].astype(np.float32)
    angle = pos / np.power(10000.0, (2.0 * np.floor(i / 2.0)) / dim)
    pe = np.zeros((seq_len, dim), dtype=np.float32)
    pe[:, 0::2] = np.sin(angle[:, 0::2])
    pe[:, 1::2] = np.cos(angle[:, 1::2])
    return pe


_K = 16
_CH = _B_PER_W // _K


_W_PER_B = _NW // BATCH
_SEQ_PER_W = SEQ // _W_PER_B


def _sc_gather_body(idx_hbm, table_hbm, out_hbm, idx_v, rows_v, sem):
    wid = lax.axis_index("s") * _NC + lax.axis_index("c")
    base = wid * _B_PER_W
    pltpu.sync_copy(idx_hbm.at[pl.ds(base, _B_PER_W)], idx_v)
    pltpu.async_copy(table_hbm.at[idx_v], rows_v, sem).wait()
    pltpu.sync_copy(rows_v, out_hbm.at[pl.ds(base, _B_PER_W)])


@functools.cache
def _sc_gather():
    return pl.kernel(
        _sc_gather_body,
        mesh=plsc.VectorSubcoreMesh(core_axis_name="c", subcore_axis_name="s"),
        compiler_params=pltpu.CompilerParams(use_tc_tiling_on_sc=False),
        out_type=jax.ShapeDtypeStruct((NTOK, FACTOR), jnp.float32),
        scratch_types=[
            pltpu.VMEM((_B_PER_W,), jnp.int32),
            pltpu.VMEM((_B_PER_W, FACTOR), jnp.float32),
            pltpu.SemaphoreType.DMA,
        ],
    )


def _leaky(x):
    return jnp.where(x >= 0, x, 0.2 * x)


def _tc_body(g_ref, pe_ref, ww_ref, wp_ref, bias_ref, gamma_ref, beta_ref, o_ref):
    w = _leaky(g_ref[...])
    p = _leaky(pe_ref[...])
    x = jnp.dot(w, ww_ref[...], preferred_element_type=jnp.float32)
    x = x + jnp.dot(p, wp_ref[...], preferred_element_type=jnp.float32)
    x = x + bias_ref[...]
    mean = jnp.mean(x, axis=-1, keepdims=True)
    xc = x - mean
    var = jnp.mean(xc * xc, axis=-1, keepdims=True)
    inv = lax.rsqrt(var + 1e-12)
    o_ref[0] = gamma_ref[...] * (xc * inv) + beta_ref[...]


S_BLK = 1024
_SB = SEQ // S_BLK


def _tc_call(gathered, pe, Ww, Wp, bias, gamma, beta):
    return pl.pallas_call(
        _tc_body,
        grid=(_SB, BATCH),
        in_specs=[
            pl.BlockSpec((S_BLK, FACTOR), lambda s, b: (b * _SB + s, 0)),
            pl.BlockSpec((S_BLK, FACTOR), lambda s, b: (s, 0)),
            pl.BlockSpec((FACTOR, HIDDEN), lambda s, b: (0, 0)),
            pl.BlockSpec((FACTOR, HIDDEN), lambda s, b: (0, 0)),
            pl.BlockSpec((1, HIDDEN), lambda s, b: (0, 0)),
            pl.BlockSpec((1, HIDDEN), lambda s, b: (0, 0)),
            pl.BlockSpec((1, HIDDEN), lambda s, b: (0, 0)),
        ],
        out_specs=pl.BlockSpec((1, S_BLK, HIDDEN), lambda s, b: (b, s, 0)),
        out_shape=jax.ShapeDtypeStruct((BATCH, SEQ, HIDDEN), jnp.float32),
    )(gathered, pe, Ww, Wp, bias, gamma, beta)


def kernel(input_tokens, emb_table, Ww, bw, Wp, bp, gamma, beta):
    idx = input_tokens.reshape(-1).astype(jnp.int32)
    gathered = _sc_gather()(idx, emb_table)
    pe = jnp.asarray(_sinusoidal_pe(SEQ, FACTOR))
    bias = (bw + bp).reshape(1, HIDDEN)
    return _tc_call(gathered, pe, Ww, Wp, bias,
                    gamma.reshape(1, HIDDEN), beta.reshape(1, HIDDEN))


# speedup vs baseline: 1.3743x; 1.3743x over previous
import functools

import jax
import jax.numpy as jnp
import numpy as np
from jax import lax
from jax.experimental import pallas as pl
from jax.experimental.pallas import tpu as pltpu
from jax.experimental.pallas import tpu_sc as plsc

VOCAB = 100000
FACTOR = 64
HIDDEN = 768
BATCH = 4
SEQ = 2048
NTOK = BATCH * SEQ

_NC, _NS = 2, 16
_NW = _NC * _NS
_B_PER_W = NTOK // _NW


def _sinusoidal_pe(seq_len, dim):
    pos = np.arange(seq_len)[:, None].astype(np.float32)
    i = np.arange(dim)[None, :].astype(np.float32)
    angle = pos / np.power(10000.0, (2.0 * np.floor(i / 2.0)) / dim)
    pe = np.zeros((seq_len, dim), dtype=np.float32)
    pe[:, 0::2] = np.sin(angle[:, 0::2])
    pe[:, 1::2] = np.cos(angle[:, 1::2])
    return pe


_W_PER_B = _NW // BATCH
_SEQ_PER_W = SEQ // _W_PER_B


_K = 16
_CH = _B_PER_W // _K


def _sc_gather_body(idx_hbm, table_hbm, out_hbm, idx_s, rows_v, sem):
    wid = lax.axis_index("s") * _NC + lax.axis_index("c")
    b = wid // _W_PER_B
    soff = (wid % _W_PER_B) * _SEQ_PER_W
    base = wid * _B_PER_W
    pltpu.sync_copy(idx_hbm.at[b, pl.ds(soff, _SEQ_PER_W)], idx_s)

    def fire(c):
        offs = c * _K
        vec = idx_s[pl.ds(offs, _K)]
        for k in range(_K):
            tok = vec[k]
            pltpu.async_copy(table_hbm.at[pl.ds(tok, 1)],
                             rows_v.at[pl.ds(offs + k, 1)], sem)

    def drain():
        for _ in range(_K):
            pltpu.make_async_copy(table_hbm.at[pl.ds(0, 1)],
                                  rows_v.at[pl.ds(0, 1)], sem).wait()

    fire(0)
    fire(1)

    @pl.loop(2, _CH)
    def _(c):
        fire(c)
        drain()

    drain()
    drain()
    pltpu.sync_copy(rows_v, out_hbm.at[pl.ds(base, _B_PER_W)])


@functools.cache
def _sc_gather():
    return pl.kernel(
        _sc_gather_body,
        mesh=plsc.VectorSubcoreMesh(core_axis_name="c", subcore_axis_name="s"),
        out_type=jax.ShapeDtypeStruct((NTOK, FACTOR), jnp.float32),
        scratch_types=[
            pltpu.VMEM((_B_PER_W,), jnp.int32),
            pltpu.VMEM((_B_PER_W, FACTOR), jnp.float32),
            pltpu.SemaphoreType.DMA,
        ],
    )


def _leaky(x):
    return jnp.where(x >= 0, x, 0.2 * x)


def _tc_body(g_ref, pe_ref, ww_ref, wp_ref, bias_ref, gamma_ref, beta_ref, o_ref):
    w = _leaky(g_ref[...])
    p = _leaky(pe_ref[...])
    x = jnp.dot(w, ww_ref[...], preferred_element_type=jnp.float32)
    x = x + jnp.dot(p, wp_ref[...], preferred_element_type=jnp.float32)
    x = x + bias_ref[...]
    mean = jnp.mean(x, axis=-1, keepdims=True)
    xc = x - mean
    var = jnp.mean(xc * xc, axis=-1, keepdims=True)
    inv = lax.rsqrt(var + 1e-12)
    o_ref[0] = gamma_ref[...] * (xc * inv) + beta_ref[...]


S_BLK = 1024
_SB = SEQ // S_BLK


def _tc_call(gathered, pe, Ww, Wp, bias, gamma, beta):
    return pl.pallas_call(
        _tc_body,
        grid=(_SB, BATCH),
        in_specs=[
            pl.BlockSpec((S_BLK, FACTOR), lambda s, b: (b * _SB + s, 0)),
            pl.BlockSpec((S_BLK, FACTOR), lambda s, b: (s, 0)),
            pl.BlockSpec((FACTOR, HIDDEN), lambda s, b: (0, 0)),
            pl.BlockSpec((FACTOR, HIDDEN), lambda s, b: (0, 0)),
            pl.BlockSpec((1, HIDDEN), lambda s, b: (0, 0)),
            pl.BlockSpec((1, HIDDEN), lambda s, b: (0, 0)),
            pl.BlockSpec((1, HIDDEN), lambda s, b: (0, 0)),
        ],
        out_specs=pl.BlockSpec((1, S_BLK, HIDDEN), lambda s, b: (b, s, 0)),
        out_shape=jax.ShapeDtypeStruct((BATCH, SEQ, HIDDEN), jnp.float32),
    )(gathered, pe, Ww, Wp, bias, gamma, beta)


def kernel(input_tokens, emb_table, Ww, bw, Wp, bp, gamma, beta):
    idx = input_tokens.astype(jnp.int32)
    gathered = _sc_gather()(idx, emb_table)
    pe = jnp.asarray(_sinusoidal_pe(SEQ, FACTOR))
    bias = (bw + bp).reshape(1, HIDDEN)
    return _tc_call(gathered, pe, Ww, Wp, bias,
                    gamma.reshape(1, HIDDEN), beta.reshape(1, HIDDEN))


# speedup vs baseline: 1.4020x; 1.0201x over previous
import functools

import jax
import jax.numpy as jnp
import numpy as np
from jax import lax
from jax.experimental import pallas as pl
from jax.experimental.pallas import tpu as pltpu
from jax.experimental.pallas import tpu_sc as plsc

VOCAB = 100000
FACTOR = 64
HIDDEN = 768
BATCH = 4
SEQ = 2048
NTOK = BATCH * SEQ

_NC, _NS = 2, 16
_NW = _NC * _NS
_B_PER_W = NTOK // _NW


def _sinusoidal_pe(seq_len, dim):
    pos = np.arange(seq_len)[:, None].astype(np.float32)
    i = np.arange(dim)[None, :].astype(np.float32)
    angle = pos / np.power(10000.0, (2.0 * np.floor(i / 2.0)) / dim)
    pe = np.zeros((seq_len, dim), dtype=np.float32)
    pe[:, 0::2] = np.sin(angle[:, 0::2])
    pe[:, 1::2] = np.cos(angle[:, 1::2])
    return pe


_W_PER_B = _NW // BATCH
_SEQ_PER_W = SEQ // _W_PER_B


_K = 16
_CH = _B_PER_W // _K


def _sc_gather_body(idx_hbm, table_hbm, out_hbm, idx_s, rows_v, sem):
    wid = lax.axis_index("s") * _NC + lax.axis_index("c")
    b = wid // _W_PER_B
    soff = (wid % _W_PER_B) * _SEQ_PER_W
    base = wid * _B_PER_W
    pltpu.sync_copy(idx_hbm.at[b, pl.ds(soff, _SEQ_PER_W)], idx_s)

    def fire(c):
        offs = c * _K
        vec = idx_s[pl.ds(offs, _K)]
        for k in range(_K):
            tok = vec[k]
            pltpu.async_copy(table_hbm.at[pl.ds(tok, 1)],
                             rows_v.at[pl.ds(offs + k, 1)], sem)

    def drain():
        for _ in range(_K):
            pltpu.make_async_copy(table_hbm.at[pl.ds(0, 1)],
                                  rows_v.at[pl.ds(0, 1)], sem).wait()

    fire(0)
    fire(1)
    fire(2)

    @pl.loop(3, _CH)
    def _(c):
        fire(c)
        drain()

    drain()
    drain()
    drain()
    pltpu.sync_copy(rows_v, out_hbm.at[pl.ds(base, _B_PER_W)])


@functools.cache
def _sc_gather():
    return pl.kernel(
        _sc_gather_body,
        mesh=plsc.VectorSubcoreMesh(core_axis_name="c", subcore_axis_name="s"),
        out_type=jax.ShapeDtypeStruct((NTOK, FACTOR), jnp.float32),
        scratch_types=[
            pltpu.VMEM((_B_PER_W,), jnp.int32),
            pltpu.VMEM((_B_PER_W, FACTOR), jnp.float32),
            pltpu.SemaphoreType.DMA,
        ],
    )


def _leaky(x):
    return jnp.where(x >= 0, x, 0.2 * x)


def _tc_body(g_ref, pe_ref, ww_ref, wp_ref, bias_ref, gamma_ref, beta_ref, o_ref):
    w = _leaky(g_ref[...])
    p = _leaky(pe_ref[...])
    x = jnp.dot(w, ww_ref[...], preferred_element_type=jnp.float32)
    x = x + jnp.dot(p, wp_ref[...], preferred_element_type=jnp.float32)
    x = x + bias_ref[...]
    mean = jnp.mean(x, axis=-1, keepdims=True)
    xc = x - mean
    var = jnp.mean(xc * xc, axis=-1, keepdims=True)
    inv = lax.rsqrt(var + 1e-12)
    o_ref[0] = gamma_ref[...] * (xc * inv) + beta_ref[...]


S_BLK = 1024
_SB = SEQ // S_BLK


def _tc_call(gathered, pe, Ww, Wp, bias, gamma, beta):
    return pl.pallas_call(
        _tc_body,
        grid=(_SB, BATCH),
        in_specs=[
            pl.BlockSpec((S_BLK, FACTOR), lambda s, b: (b * _SB + s, 0)),
            pl.BlockSpec((S_BLK, FACTOR), lambda s, b: (s, 0)),
            pl.BlockSpec((FACTOR, HIDDEN), lambda s, b: (0, 0)),
            pl.BlockSpec((FACTOR, HIDDEN), lambda s, b: (0, 0)),
            pl.BlockSpec((1, HIDDEN), lambda s, b: (0, 0)),
            pl.BlockSpec((1, HIDDEN), lambda s, b: (0, 0)),
            pl.BlockSpec((1, HIDDEN), lambda s, b: (0, 0)),
        ],
        out_specs=pl.BlockSpec((1, S_BLK, HIDDEN), lambda s, b: (b, s, 0)),
        out_shape=jax.ShapeDtypeStruct((BATCH, SEQ, HIDDEN), jnp.float32),
    )(gathered, pe, Ww, Wp, bias, gamma, beta)


def kernel(input_tokens, emb_table, Ww, bw, Wp, bp, gamma, beta):
    idx = input_tokens.astype(jnp.int32)
    gathered = _sc_gather()(idx, emb_table)
    pe = jnp.asarray(_sinusoidal_pe(SEQ, FACTOR))
    bias = (bw + bp).reshape(1, HIDDEN)
    return _tc_call(gathered, pe, Ww, Wp, bias,
                    gamma.reshape(1, HIDDEN), beta.reshape(1, HIDDEN))
